# baseline (speedup 1.0000x reference)
import functools

import jax
import jax.numpy as jnp
from jax import lax
from jax.experimental import pallas as pl
from jax.experimental.pallas import tpu as pltpu
from jax.experimental.pallas import tpu_sc as plsc

N_NODES = 10000
N_EDGES = 320000
F_IN = 128
F_HID = 512

NC = 2
NS = 16
NW = NC * NS
CHUNK = 128
CPW = 80
EW = CPW * CHUNK
NSLOT = 4
E_PAD = EW * NW
N_PAD = 10240
RPS = N_PAD // NS

BLK = 1024
GRID = N_PAD // BLK


_mesh = plsc.VectorSubcoreMesh(core_axis_name="c", subcore_axis_name="s")


def _f32(shape):
    return jax.ShapeDtypeStruct(shape, jnp.float32)


def _degrees(src, dst, zrows, ones_rows):
    @functools.partial(
        pl.kernel,
        out_type=[_f32((NC, N_PAD, F_IN)), _f32((NC, N_PAD, F_IN))],
        mesh=_mesh,
        scratch_types=[
            pltpu.VMEM_SHARED((N_PAD, F_IN), jnp.float32),
            pltpu.VMEM((CHUNK, F_IN), jnp.float32),
            pltpu.VMEM((CPW, CHUNK), jnp.int32),
            pltpu.VMEM((CPW, CHUNK), jnp.int32),
            pltpu.SemaphoreType.DMA,
        ],
    )
    def k(src_h, dst_h, z_h, ones_h, dego_h, degi_h, acc_sh, ones_v, sidx2,
          didx2, sem):
        c = lax.axis_index("c")
        s = lax.axis_index("s")
        cbase = (c * NS + s) * CPW
        rbase = s * RPS
        pltpu.sync_copy(ones_h, ones_v)
        pltpu.sync_copy(src_h.at[pl.ds(cbase, CPW)], sidx2)
        pltpu.sync_copy(dst_h.at[pl.ds(cbase, CPW)], didx2)
        for idx2, out_ref in ((sidx2, dego_h), (didx2, degi_h)):
            pltpu.sync_copy(z_h, acc_sh.at[pl.ds(rbase, RPS)])
            plsc.subcore_barrier()

            @pl.loop(0, CPW)
            def _(j, idx2=idx2):
                pltpu.sync_copy(ones_v, acc_sh.at[idx2.at[j]], add=True)

            plsc.subcore_barrier()
            pltpu.sync_copy(acc_sh.at[pl.ds(rbase, RPS)],
                            out_ref.at[c, pl.ds(rbase, RPS)])

    return k(src, dst, zrows, ones_rows)


def _propagate(table, src, dst, zrows):
    @functools.partial(
        pl.kernel,
        out_type=_f32((NC, N_PAD, F_IN)),
        mesh=_mesh,
        scratch_types=(
            [pltpu.VMEM_SHARED((N_PAD, F_IN), jnp.float32)]
            + [pltpu.VMEM((CHUNK, F_IN), jnp.float32)]
            + [pltpu.VMEM((CHUNK,), jnp.int32)] * 2
            + [pltpu.SemaphoreType.DMA]
        ),
    )
    def k(tab_h, src_h, dst_h, z_h, out_h, acc_sh, rows_v, sidx, didx, sg):
        c = lax.axis_index("c")
        s = lax.axis_index("s")
        rbase = s * RPS
        base = (c * NS + s) * EW
        pltpu.sync_copy(z_h, acc_sh.at[pl.ds(rbase, RPS)])
        plsc.subcore_barrier()

        @pl.loop(0, CPW)
        def _(j):
            pltpu.sync_copy(src_h.at[pl.ds(base + j * CHUNK, CHUNK)], sidx)
            pltpu.sync_copy(dst_h.at[pl.ds(base + j * CHUNK, CHUNK)], didx)
            pltpu.async_copy(tab_h.at[sidx], rows_v, sg).wait()
            pltpu.sync_copy(rows_v, acc_sh.at[didx], add=True)

        plsc.subcore_barrier()
        pltpu.sync_copy(acc_sh.at[pl.ds(rbase, RPS)],
                        out_h.at[c, pl.ds(rbase, RPS)])

    return k(table, src, dst, zrows)


def _norm_from_deg(deg):
    return jnp.where(deg > 0, lax.rsqrt(jnp.maximum(deg, 1.0)), 0.0)


def _norm_h0_body(x_ref, dego_ref, degi_ref, h0_ref, ns_ref, nd_ref):
    deg_o = dego_ref[0] + dego_ref[1]
    deg_i = degi_ref[0] + degi_ref[1]
    row = lax.broadcasted_iota(jnp.int32, (N_PAD, 1), 0)
    valid = (row < N_NODES).astype(jnp.float32)
    ns = _norm_from_deg(deg_o) * valid
    nd = _norm_from_deg(deg_i) * valid
    ns_ref[...] = ns
    nd_ref[...] = nd
    h0_ref[...] = x_ref[...] * ns


def _norm_h0(x_pad, dego, degi):
    return pl.pallas_call(
        _norm_h0_body,
        out_shape=[_f32((N_PAD, F_IN)), _f32((N_PAD, 1)), _f32((N_PAD, 1))],
    )(x_pad, dego, degi)


def _mm_body(p_ref, ns_ref, nd_ref, w1_ref, b1_ref, w2_ref, g_ref):
    p = (p_ref[0] + p_ref[1]) * nd_ref[...]
    h1 = jnp.dot(p, w1_ref[...], preferred_element_type=jnp.float32,
                 precision=lax.Precision.HIGHEST)
    h1 = jnp.maximum(h1 + b1_ref[...], 0.0) * ns_ref[...]
    g_ref[...] = jnp.dot(h1, w2_ref[...], preferred_element_type=jnp.float32,
                         precision=lax.Precision.HIGHEST)


def _mm(p1, ns, nd, W1, b1, W2):
    return pl.pallas_call(
        _mm_body,
        grid=(GRID,),
        in_specs=[
            pl.BlockSpec((NC, BLK, F_IN), lambda i: (0, i, 0)),
            pl.BlockSpec((BLK, 1), lambda i: (i, 0)),
            pl.BlockSpec((BLK, 1), lambda i: (i, 0)),
            pl.BlockSpec((F_IN, F_HID), lambda i: (0, 0)),
            pl.BlockSpec((1, F_HID), lambda i: (0, 0)),
            pl.BlockSpec((F_HID, F_IN), lambda i: (0, 0)),
        ],
        out_specs=pl.BlockSpec((BLK, F_IN), lambda i: (i, 0)),
        out_shape=_f32((N_PAD, F_IN)),
    )(p1, ns, nd, W1, b1, W2)


def _fin_body(q_ref, nd_ref, b2_ref, o_ref):
    o_ref[...] = (q_ref[0] + q_ref[1]) * nd_ref[...] + b2_ref[...]


def _fin(p2, nd, b2):
    return pl.pallas_call(
        _fin_body,
        grid=(GRID,),
        in_specs=[
            pl.BlockSpec((NC, BLK, F_IN), lambda i: (0, i, 0)),
            pl.BlockSpec((BLK, 1), lambda i: (i, 0)),
            pl.BlockSpec((1, F_IN), lambda i: (0, 0)),
        ],
        out_specs=pl.BlockSpec((BLK, F_IN), lambda i: (i, 0)),
        out_shape=_f32((N_PAD, F_IN)),
    )(p2, nd, b2)


@jax.jit
def kernel(x, edge_index, W1, b1, W2, b2):
    src = edge_index[0].astype(jnp.int32)
    dst = edge_index[1].astype(jnp.int32)
    pad = jnp.full((E_PAD - N_EDGES,), N_NODES, jnp.int32)
    src_p = jnp.concatenate([src, pad]).reshape(NW * CPW, CHUNK)
    dst_p = jnp.concatenate([dst, pad]).reshape(NW * CPW, CHUNK)
    x_p = jnp.pad(x, ((0, N_PAD - N_NODES), (0, 0)))

    ones_rows = jnp.ones((CHUNK, F_IN), jnp.float32)
    zfeat = jnp.zeros((RPS, F_IN), jnp.float32)

    dego, degi = _degrees(src_p, dst_p, zfeat, ones_rows)
    h0, ns, nd = _norm_h0(x_p, dego[:, :, 0:1], degi[:, :, 0:1])
    src_f = src_p.reshape(E_PAD)
    dst_f = dst_p.reshape(E_PAD)
    p1 = _propagate(h0, src_f, dst_f, zfeat)
    g = _mm(p1, ns, nd, W1, b1.reshape(1, F_HID), W2)
    p2 = _propagate(g, src_f, dst_f, zfeat)
    out = _fin(p2, nd, b2.reshape(1, F_IN))
    return out[:N_NODES]


# speedup vs baseline: 1.4575x; 1.4575x over previous
import functools

import jax
import jax.numpy as jnp
from jax import lax
from jax.experimental import pallas as pl
from jax.experimental.pallas import tpu as pltpu
from jax.experimental.pallas import tpu_sc as plsc

N_NODES = 10000
N_EDGES = 320000
F_IN = 128
F_HID = 512

NC = 2
NS = 16
NW = NC * NS
CHUNK = 128
CPW = 80
EW = CPW * CHUNK
NSLOT = 4
CPW_P0 = 120
CPW_P1 = 2 * CPW - CPW_P0
E_PAD = EW * NW
N_PAD = 10240
RPS = N_PAD // NS

BLK = 1024
GRID = N_PAD // BLK


_mesh = plsc.VectorSubcoreMesh(core_axis_name="c", subcore_axis_name="s")


def _f32(shape):
    return jax.ShapeDtypeStruct(shape, jnp.float32)


def _degrees(src, dst, zrows, ones_rows):
    @functools.partial(
        pl.kernel,
        out_type=[_f32((NC, N_PAD, F_IN)), _f32((NC, N_PAD, F_IN))],
        mesh=_mesh,
        scratch_types=[
            pltpu.VMEM_SHARED((N_PAD, F_IN), jnp.float32),
            pltpu.VMEM((CHUNK, F_IN), jnp.float32),
            pltpu.VMEM((CPW, CHUNK), jnp.int32),
            pltpu.VMEM((CPW, CHUNK), jnp.int32),
            pltpu.SemaphoreType.DMA,
        ],
    )
    def k(src_h, dst_h, z_h, ones_h, dego_h, degi_h, acc_sh, ones_v, sidx2,
          didx2, sem):
        c = lax.axis_index("c")
        s = lax.axis_index("s")
        cbase = (c * NS + s) * CPW
        rbase = s * RPS
        pltpu.sync_copy(ones_h, ones_v)
        pltpu.sync_copy(src_h.at[pl.ds(cbase, CPW)], sidx2)
        pltpu.sync_copy(dst_h.at[pl.ds(cbase, CPW)], didx2)
        for idx2, out_ref in ((sidx2, dego_h), (didx2, degi_h)):
            pltpu.sync_copy(z_h, acc_sh.at[pl.ds(rbase, RPS)])
            plsc.subcore_barrier()

            @pl.loop(0, CPW)
            def _(j, idx2=idx2):
                pltpu.sync_copy(ones_v, acc_sh.at[idx2.at[j]], add=True)

            plsc.subcore_barrier()
            pltpu.sync_copy(acc_sh.at[pl.ds(rbase, RPS)],
                            out_ref.at[c, pl.ds(rbase, RPS)])

    return k(src, dst, zrows, ones_rows)


def _propagate(table, src, dst, zrows):
    @functools.partial(
        pl.kernel,
        out_type=_f32((NC, N_PAD, F_IN)),
        mesh=_mesh,
        scratch_types=(
            [pltpu.VMEM_SHARED((N_PAD, F_IN), jnp.float32)]
            + [pltpu.VMEM((CHUNK, F_IN), jnp.float32)] * 2
            + [pltpu.VMEM((CPW_P0, CHUNK), jnp.int32)]
            + [pltpu.VMEM((1, CHUNK), jnp.int32)] * 2
            + [pltpu.SemaphoreType.DMA] * 6
        ),
    )
    def k(tab_h, src_h, dst_h, z_h, out_h, acc_sh, r0, r1, didx2, s0, s1,
          sg0, sg1, ss0, ss1, si0, si1):
        rows = (r0, r1)
        sbuf = (s0, s1)
        sg = (sg0, sg1)
        ss = (ss0, ss1)
        si = (si0, si1)
        c = lax.axis_index("c")
        s = lax.axis_index("s")
        cbase = jnp.where(c == 0, s * CPW_P0, NS * CPW_P0 + s * CPW_P1)
        cpw = jnp.where(c == 0, CPW_P0, CPW_P1)
        rbase = s * RPS

        @pl.when(c == 0)
        def _():
            pltpu.sync_copy(dst_h.at[pl.ds(cbase, CPW_P0)],
                            didx2.at[pl.ds(0, CPW_P0)])

        @pl.when(c == 1)
        def _():
            pltpu.sync_copy(dst_h.at[pl.ds(cbase, CPW_P1)],
                            didx2.at[pl.ds(0, CPW_P1)])

        pltpu.sync_copy(src_h.at[pl.ds(cbase, 1)], sbuf[0])
        pltpu.async_copy(tab_h.at[sbuf[0].at[0]], rows[0], sg[0])
        pltpu.async_copy(src_h.at[pl.ds(cbase + 1, 1)], sbuf[1], si[1])
        pltpu.sync_copy(z_h, acc_sh.at[pl.ds(rbase, RPS)])
        plsc.subcore_barrier()

        @pl.loop(0, cpw // 2)
        def _(g):
            for b in range(2):
                j = 2 * g + b
                o = 1 - b

                @pl.when(j > 0)
                def _(b=b, j=j, o=o):
                    pltpu.make_async_copy(rows[o], acc_sh.at[didx2.at[0]],
                                          ss[o]).wait()

                @pl.when(j + 1 < cpw)
                def _(b=b, j=j, o=o):
                    pltpu.make_async_copy(src_h.at[pl.ds(cbase, 1)], sbuf[o],
                                          si[o]).wait()
                    pltpu.async_copy(tab_h.at[sbuf[o].at[0]], rows[o], sg[o])

                pltpu.make_async_copy(tab_h.at[sbuf[b].at[0]], rows[b],
                                      sg[b]).wait()
                pltpu.async_copy(rows[b], acc_sh.at[didx2.at[j]], ss[b],
                                 add=True)

                @pl.when(j + 2 < cpw)
                def _(b=b, j=j):
                    pltpu.async_copy(src_h.at[pl.ds(cbase + j + 2, 1)], sbuf[b],
                                     si[b])

        pltpu.make_async_copy(rows[1], acc_sh.at[didx2.at[0]], ss[1]).wait()
        plsc.subcore_barrier()
        pltpu.sync_copy(acc_sh.at[pl.ds(rbase, RPS)],
                        out_h.at[c, pl.ds(rbase, RPS)])

    return k(table, src, dst, zrows)


def _norm_from_deg(deg):
    return jnp.where(deg > 0, lax.rsqrt(jnp.maximum(deg, 1.0)), 0.0)


def _norm_h0_body(x_ref, dego_ref, degi_ref, h0_ref, ns_ref, nd_ref):
    deg_o = dego_ref[0] + dego_ref[1]
    deg_i = degi_ref[0] + degi_ref[1]
    row = lax.broadcasted_iota(jnp.int32, (N_PAD, 1), 0)
    valid = (row < N_NODES).astype(jnp.float32)
    ns = _norm_from_deg(deg_o) * valid
    nd = _norm_from_deg(deg_i) * valid
    ns_ref[...] = ns
    nd_ref[...] = nd
    h0_ref[...] = x_ref[...] * ns


def _norm_h0(x_pad, dego, degi):
    return pl.pallas_call(
        _norm_h0_body,
        out_shape=[_f32((N_PAD, F_IN)), _f32((N_PAD, 1)), _f32((N_PAD, 1))],
    )(x_pad, dego, degi)


def _mm_body(p_ref, ns_ref, nd_ref, w1_ref, b1_ref, w2_ref, g_ref):
    p = (p_ref[0] + p_ref[1]) * nd_ref[...]
    h1 = jnp.dot(p, w1_ref[...], preferred_element_type=jnp.float32,
                 precision=lax.Precision.HIGHEST)
    h1 = jnp.maximum(h1 + b1_ref[...], 0.0) * ns_ref[...]
    g_ref[...] = jnp.dot(h1, w2_ref[...], preferred_element_type=jnp.float32,
                         precision=lax.Precision.HIGHEST)


def _mm(p1, ns, nd, W1, b1, W2):
    return pl.pallas_call(
        _mm_body,
        grid=(GRID,),
        in_specs=[
            pl.BlockSpec((NC, BLK, F_IN), lambda i: (0, i, 0)),
            pl.BlockSpec((BLK, 1), lambda i: (i, 0)),
            pl.BlockSpec((BLK, 1), lambda i: (i, 0)),
            pl.BlockSpec((F_IN, F_HID), lambda i: (0, 0)),
            pl.BlockSpec((1, F_HID), lambda i: (0, 0)),
            pl.BlockSpec((F_HID, F_IN), lambda i: (0, 0)),
        ],
        out_specs=pl.BlockSpec((BLK, F_IN), lambda i: (i, 0)),
        out_shape=_f32((N_PAD, F_IN)),
    )(p1, ns, nd, W1, b1, W2)


def _fin_body(q_ref, nd_ref, b2_ref, o_ref):
    o_ref[...] = (q_ref[0] + q_ref[1]) * nd_ref[...] + b2_ref[...]


def _fin(p2, nd, b2):
    return pl.pallas_call(
        _fin_body,
        grid=(GRID,),
        in_specs=[
            pl.BlockSpec((NC, BLK, F_IN), lambda i: (0, i, 0)),
            pl.BlockSpec((BLK, 1), lambda i: (i, 0)),
            pl.BlockSpec((1, F_IN), lambda i: (0, 0)),
        ],
        out_specs=pl.BlockSpec((BLK, F_IN), lambda i: (i, 0)),
        out_shape=_f32((N_PAD, F_IN)),
    )(p2, nd, b2)


@jax.jit
def kernel(x, edge_index, W1, b1, W2, b2):
    src = edge_index[0].astype(jnp.int32)
    dst = edge_index[1].astype(jnp.int32)
    pad = jnp.full((E_PAD - N_EDGES,), N_NODES, jnp.int32)
    src_p = jnp.concatenate([src, pad]).reshape(NW * CPW, CHUNK)
    dst_p = jnp.concatenate([dst, pad]).reshape(NW * CPW, CHUNK)
    x_p = jnp.pad(x, ((0, N_PAD - N_NODES), (0, 0)))

    ones_rows = jnp.ones((CHUNK, F_IN), jnp.float32)
    zfeat = jnp.zeros((RPS, F_IN), jnp.float32)

    dego, degi = _degrees(src_p, dst_p, zfeat, ones_rows)
    h0, ns, nd = _norm_h0(x_p, dego[:, :, 0:1], degi[:, :, 0:1])
    p1 = _propagate(h0, src_p, dst_p, zfeat)
    g = _mm(p1, ns, nd, W1, b1.reshape(1, F_HID), W2)
    p2 = _propagate(g, src_p, dst_p, zfeat)
    out = _fin(p2, nd, b2.reshape(1, F_IN))
    return out[:N_NODES]
